# baseline (speedup 1.0000x reference)
import functools

import jax
import jax.numpy as jnp
from jax import lax
from jax.experimental import pallas as pl
from jax.experimental.pallas import tpu as pltpu
from jax.experimental.pallas import tpu_sc as plsc

B, D, H, W = 16, 10, 128, 128
M = 500
HW = H * W
MP = 512
NW = 32
CHUNK = 256
MH = M // 2
NV = CHUNK // 16
NG = D * CHUNK
GB = 512
NS = NG // GB
TSZ = D * CHUNK

_mesh = plsc.VectorSubcoreMesh(core_axis_name="c", subcore_axis_name="s")


@functools.partial(
    pl.kernel,
    out_type=jax.ShapeDtypeStruct((NW, 12, 16), jnp.float32),
    mesh=_mesh,
    compiler_params=pltpu.CompilerParams(needs_layout_passes=False),
    scratch_types=[
        pltpu.VMEM((CHUNK,), jnp.float32),
        pltpu.VMEM((TSZ,), jnp.float32),
        pltpu.VMEM((NV, 16), jnp.float32),
        pltpu.VMEM((NG,), jnp.int32),
        pltpu.VMEM((NG,), jnp.float32),
        pltpu.VMEM((12, 16), jnp.float32),
        pltpu.SemaphoreType.DMA,
        pltpu.SemaphoreType.DMA,
    ],
)
def _sc_partials(flat_hbm, im_hbm, tgt_hbm, out_hbm,
                 imv, tv, mbuf, idx2, pred2, part, sem, sem2):
    wid = lax.axis_index("c") * 16 + lax.axis_index("s")
    b = wid // 2
    a = pl.multiple_of(wid * CHUNK, 256)
    zeros = jnp.zeros((16,), jnp.float32)
    ts = pl.multiple_of(wid * TSZ, 256)
    tcopy = pltpu.async_copy(tgt_hbm.at[pl.ds(ts, TSZ)],
                             tv.at[pl.ds(0, TSZ)], sem2)
    pltpu.sync_copy(im_hbm.at[pl.ds(a, CHUNK)], imv)
    nacc = zeros
    gcopies = [None] * NS
    for i in range(NV):
        pk = imv[pl.ds(i * 16, 16)].astype(jnp.int32)
        v = pk & (HW - 1)
        for d in range(D):
            p = d * CHUNK + i * 16
            idx2[pl.ds(p, 16)] = v + (b * D + d) * HW
        mvec = lax.shift_right_logical(pk, 14).astype(jnp.float32)
        nacc = nacc + mvec
        mbuf[i, :] = mvec
    for g in range(NS):
        gcopies[g] = pltpu.async_copy(
            flat_hbm.at[idx2.at[pl.ds(g * GB, GB)]],
            pred2.at[pl.ds(g * GB, GB)], sem)
    tcopy.wait()
    patt = lax.iota(jnp.int32, 16) * D
    dacc = [zeros for _ in range(D)]
    for g in range(NS):
        gcopies[g].wait()
        for q in range(GB // 16):
            p = g * GB + q * 16
            d = p // CHUNK
            i = (p % CHUNK) // 16
            pv = pred2[pl.ds(p, 16)]
            t = plsc.load_gather(tv, [patt + (i * 16 * D + d)])
            dacc[d] = dacc[d] + jnp.abs(pv - t) * mbuf[i, :]
    for d in range(D):
        part[d, :] = dacc[d]
    part[10, :] = nacc
    part[11, :] = zeros
    pltpu.sync_copy(part, out_hbm.at[wid])


def _finish(p_ref, o_ref):
    x = p_ref[...]
    s = jnp.sum(x, axis=(0, 2))
    o_ref[...] = s[:10] / (s[10] + 1e-4)


@jax.jit
def kernel(output, mask, ind, target):
    flat = output.reshape(B * D * HW)
    pad = ((0, 0), (0, MP - M))
    packed = (jnp.pad(ind.astype(jnp.int32), pad)
              + jnp.pad(mask.astype(jnp.int32), pad) * HW).astype(jnp.float32)
    tgt_p = jnp.pad(target, (pad[0], pad[1], (0, 0)))
    parts = _sc_partials(flat, packed.reshape(B * MP),
                         tgt_p.reshape(B * MP * D))
    return pl.pallas_call(
        _finish,
        out_shape=jax.ShapeDtypeStruct((10,), jnp.float32),
    )(parts)


# speedup vs baseline: 1.0244x; 1.0244x over previous
import functools

import jax
import jax.numpy as jnp
from jax import lax
from jax.experimental import pallas as pl
from jax.experimental.pallas import tpu as pltpu
from jax.experimental.pallas import tpu_sc as plsc

B, D, H, W = 16, 10, 128, 128
M = 500
HW = H * W
MP = 512
NW = 32
CHUNK = 256
MH = M // 2
NV = CHUNK // 16
NG = D * CHUNK
GB = 512
NS = NG // GB
DS = 11
TSZ = DS * CHUNK

_mesh = plsc.VectorSubcoreMesh(core_axis_name="c", subcore_axis_name="s")


@functools.partial(
    pl.kernel,
    out_type=jax.ShapeDtypeStruct((NW, 12, 16), jnp.float32),
    mesh=_mesh,
    compiler_params=pltpu.CompilerParams(needs_layout_passes=False),
    scratch_types=[
        pltpu.VMEM((CHUNK,), jnp.float32),
        pltpu.VMEM((TSZ,), jnp.float32),
        pltpu.VMEM((NV, 16), jnp.float32),
        pltpu.VMEM((NG,), jnp.int32),
        pltpu.VMEM((NG,), jnp.float32),
        pltpu.VMEM((12, 16), jnp.float32),
        pltpu.SemaphoreType.DMA,
        pltpu.SemaphoreType.DMA,
    ],
)
def _sc_partials(flat_hbm, im_hbm, tgt_hbm, out_hbm,
                 imv, tv, mbuf, idx2, pred2, part, sem, sem2):
    wid = lax.axis_index("c") * 16 + lax.axis_index("s")
    b = wid // 2
    a = pl.multiple_of(wid * CHUNK, 256)
    zeros = jnp.zeros((16,), jnp.float32)
    ts = pl.multiple_of(wid * TSZ, 256)
    tcopy = pltpu.async_copy(tgt_hbm.at[pl.ds(ts, TSZ)],
                             tv.at[pl.ds(0, TSZ)], sem2)
    pltpu.sync_copy(im_hbm.at[pl.ds(a, CHUNK)], imv)
    nacc = zeros
    gcopies = [None] * NS
    for i in range(NV):
        pk = imv[pl.ds(i * 16, 16)].astype(jnp.int32)
        v = pk & (HW - 1)
        for d in range(D):
            p = d * CHUNK + i * 16
            idx2[pl.ds(p, 16)] = v + (b * D + d) * HW
        mvec = lax.shift_right_logical(pk, 14).astype(jnp.float32)
        nacc = nacc + mvec
        mbuf[i, :] = mvec
    for g in range(NS):
        gcopies[g] = pltpu.async_copy(
            flat_hbm.at[idx2.at[pl.ds(g * GB, GB)]],
            pred2.at[pl.ds(g * GB, GB)], sem)
    tcopy.wait()
    patt = lax.iota(jnp.int32, 16) * DS
    dacc = [zeros for _ in range(D)]
    for g in range(NS):
        gcopies[g].wait()
        for q in range(GB // 16):
            p = g * GB + q * 16
            d = p // CHUNK
            i = (p % CHUNK) // 16
            pv = pred2[pl.ds(p, 16)]
            t = plsc.load_gather(tv, [patt + (i * 16 * DS + d)])
            dacc[d] = dacc[d] + jnp.abs(pv - t) * mbuf[i, :]
    for d in range(D):
        part[d, :] = dacc[d]
    part[10, :] = nacc
    part[11, :] = zeros
    pltpu.sync_copy(part, out_hbm.at[wid])


def _finish(p_ref, o_ref):
    x = p_ref[...]
    s = jnp.sum(x, axis=(0, 2))
    o_ref[...] = s[:10] / (s[10] + 1e-4)


@jax.jit
def kernel(output, mask, ind, target):
    flat = output.reshape(B * D * HW)
    pad = ((0, 0), (0, MP - M))
    packed = (jnp.pad(ind.astype(jnp.int32), pad)
              + jnp.pad(mask.astype(jnp.int32), pad) * HW).astype(jnp.float32)
    tgt_p = jnp.pad(target, (pad[0], pad[1], (0, DS - D)))
    parts = _sc_partials(flat, packed.reshape(B * MP),
                         tgt_p.reshape(B * MP * DS))
    return pl.pallas_call(
        _finish,
        out_shape=jax.ShapeDtypeStruct((10,), jnp.float32),
    )(parts)


# speedup vs baseline: 1.2883x; 1.2577x over previous
import functools

import jax
import jax.numpy as jnp
from jax import lax
from jax.experimental import pallas as pl
from jax.experimental.pallas import tpu as pltpu
from jax.experimental.pallas import tpu_sc as plsc

B, D, H, W = 16, 10, 128, 128
M = 500
HW = H * W
MP = 512
NW = 32
CHUNK = 256
NV = CHUNK // 16
NG = D * CHUNK
GB = 512
NS = NG // GB

_mesh = plsc.VectorSubcoreMesh(core_axis_name="c", subcore_axis_name="s")


@functools.partial(
    pl.kernel,
    out_type=jax.ShapeDtypeStruct((NW, 12, 16), jnp.float32),
    mesh=_mesh,
    scratch_types=[
        pltpu.VMEM((CHUNK,), jnp.float32),
        pltpu.VMEM((D * CHUNK,), jnp.float32),
        pltpu.VMEM((NV, 16), jnp.float32),
        pltpu.VMEM((NG,), jnp.int32),
        pltpu.VMEM((NG,), jnp.float32),
        pltpu.VMEM((12, 16), jnp.float32),
        pltpu.SemaphoreType.DMA,
        pltpu.SemaphoreType.DMA,
    ],
)
def _sc_partials(flat_hbm, im_hbm, tgt_hbm, out_hbm,
                 imv, tv, mbuf, idx2, pred2, part, sem, sem2):
    wid = lax.axis_index("c") * 16 + lax.axis_index("s")
    b = wid // 2
    a = pl.multiple_of(wid * CHUNK, 256)
    pltpu.sync_copy(im_hbm.at[pl.ds(a, CHUNK)], imv)
    tcopies = [
        pltpu.async_copy(tgt_hbm.at[pl.ds(d * B * MP + a, CHUNK)],
                         tv.at[pl.ds(d * CHUNK, CHUNK)], sem2)
        for d in range(D)
    ]
    zeros = jnp.zeros((16,), jnp.float32)
    nacc = zeros
    gcopies = [None] * NS
    for i in range(NV):
        pk = imv[pl.ds(i * 16, 16)].astype(jnp.int32)
        v = pk & (HW - 1)
        for d in range(D):
            p = d * CHUNK + i * 16
            idx2[pl.ds(p, 16)] = v + (b * D + d) * HW
        mvec = lax.shift_right_logical(pk, 14).astype(jnp.float32)
        nacc = nacc + mvec
        mbuf[i, :] = mvec
    for g in range(NS):
        gcopies[g] = pltpu.async_copy(
            flat_hbm.at[idx2.at[pl.ds(g * GB, GB)]],
            pred2.at[pl.ds(g * GB, GB)], sem)
    for c in tcopies:
        c.wait()
    dacc = [zeros for _ in range(D)]
    for g in range(NS):
        gcopies[g].wait()
        for q in range(GB // 16):
            p = g * GB + q * 16
            d = p // CHUNK
            i = (p % CHUNK) // 16
            pv = pred2[pl.ds(p, 16)]
            t = tv[pl.ds(p, 16)]
            dacc[d] = dacc[d] + jnp.abs(pv - t) * mbuf[i, :]
    for d in range(D):
        part[d, :] = dacc[d]
    part[10, :] = nacc
    part[11, :] = zeros
    pltpu.sync_copy(part, out_hbm.at[wid])


def _finish(p_ref, o_ref):
    x = p_ref[...]
    s = jnp.sum(x, axis=(0, 2))
    o_ref[...] = s[:10] / (s[10] + 1e-4)


@jax.jit
def kernel(output, mask, ind, target):
    flat = output.reshape(B * D * HW)
    pad = ((0, 0), (0, MP - M))
    packed = (jnp.pad(ind.astype(jnp.int32), pad)
              + jnp.pad(mask.astype(jnp.int32), pad) * HW).astype(jnp.float32)
    tgt_t = jnp.pad(target, (pad[0], pad[1], (0, 0))).transpose(2, 0, 1)
    parts = _sc_partials(flat, packed.reshape(B * MP),
                         tgt_t.reshape(D * B * MP))
    return pl.pallas_call(
        _finish,
        out_shape=jax.ShapeDtypeStruct((10,), jnp.float32),
    )(parts)
